# baseline (speedup 1.0000x reference)
import functools

import jax
import jax.numpy as jnp
from jax import lax
from jax.experimental import pallas as pl
from jax.experimental.pallas import tpu as pltpu
from jax.experimental.pallas import tpu_sc as plsc

VOCAB = 1000000
EMBED_DIM = 64
BATCH = 16384
HIST_LEN = 50

_B = BATCH * HIST_LEN
_NW = 32
_NBLK = _B // 128
_BLK_PER_W = _NBLK // _NW
_NBUF = 2

_mesh = plsc.VectorSubcoreMesh(core_axis_name="c", subcore_axis_name="s")


@functools.partial(
    pl.kernel,
    mesh=_mesh,
    out_type=jax.ShapeDtypeStruct((HIST_LEN, 8, 128, 8, 128), jnp.float32),
    scratch_types=[
        pltpu.VMEM((_BLK_PER_W, 128), jnp.int32),
        pltpu.VMEM((_NBUF, 128), jnp.int32),
        pltpu.VMEM((_NBUF, 128), jnp.int32),
        [pltpu.VMEM((128, 128), jnp.float32) for _ in range(_NBUF)],
        [pltpu.VMEM((64, 128), jnp.float32) for _ in range(_NBUF)],
        [pltpu.SemaphoreType.DMA for _ in range(_NBUF)],
        [pltpu.SemaphoreType.DMA for _ in range(_NBUF)],
    ],
    compiler_params=pltpu.CompilerParams(
        use_tc_tiling_on_sc=True, needs_layout_passes=False),
)
def _lookup_kernel(table_hbm, idx_hbm, out_hbm, idx_v, sr_v, colb_v,
                   staged, tout, gsems, ssems):
    wid = lax.axis_index("s") * 2 + lax.axis_index("c")
    blk0 = wid * _BLK_PER_W
    pltpu.sync_copy(idx_hbm.at[pl.ds(blk0, _BLK_PER_W)], idx_v)

    iotas = [lax.iota(jnp.int32, 16) + (16 * g) for g in range(8)]

    def prep(t, p):
        for g in range(8):
            v = idx_v[t, pl.ds(16 * g, 16)]
            sr_v[p, pl.ds(16 * g, 16)] = lax.shift_right_logical(v, 1)
            colb_v[p, pl.ds(16 * g, 16)] = lax.shift_left(
                lax.bitwise_and(v, 1), 6)

    def fire_gather(p):
        pltpu.async_copy(table_hbm.at[sr_v.at[p]], staged[p], gsems[p])

    def wait_gather(p):
        pltpu.make_async_copy(table_hbm.at[pl.ds(0, 128)], staged[p],
                              gsems[p]).wait()

    def wait_stores(p):
        pltpu.make_async_copy(table_hbm.at[pl.ds(0, 32)], tout[p],
                              ssems[p]).wait()

    def transpose(p):
        colbs = [colb_v[p, pl.ds(16 * g, 16)] for g in range(8)]

        def frow(f, carry):
            for g in range(8):
                vals = plsc.load_gather(staged[p], [iotas[g], colbs[g] + f])
                tout[p][f, pl.ds(16 * g, 16)] = vals
            return carry

        lax.fori_loop(0, 64, frow, 0)

    def fire_stores(t, p):
        blk = blk0 + t
        h = blk // 128
        bg = lax.rem(blk, 128)
        for fg in range(8):
            pltpu.async_copy(tout[p].at[pl.ds(8 * fg, 8)],
                             out_hbm.at[h, fg, bg], ssems[p])

    for p in range(_NBUF):
        prep(p, p)
        fire_gather(p)

    for p in range(_NBUF):
        wait_gather(p)
        transpose(p)
        fire_stores(p, p)
        prep(p + _NBUF, p)
        fire_gather(p)

    def outer(t, carry):
        for p in range(_NBUF):
            blk = t * _NBUF + p
            wait_gather(p)
            wait_stores(p)
            transpose(p)
            fire_stores(blk, p)
            prep(blk + _NBUF, p)
            fire_gather(p)
        return carry

    lax.fori_loop(1, _BLK_PER_W // _NBUF - 1, outer, 0, unroll=False)

    for p in range(_NBUF):
        blk = _BLK_PER_W - _NBUF + p
        wait_gather(p)
        wait_stores(p)
        transpose(p)
        fire_stores(blk, p)
    for p in range(_NBUF):
        wait_stores(p)


def kernel(item_embeddings, batch_data):
    table = item_embeddings.reshape(VOCAB // 2, 2 * EMBED_DIM)
    idx = batch_data.T.astype(jnp.int32).reshape(_NBLK, 128)
    out5d = _lookup_kernel(table, idx)
    out = out5d.transpose(2, 4, 0, 1, 3).reshape(BATCH, HIST_LEN, EMBED_DIM)
    return out
